# Initial kernel scaffold; baseline (speedup 1.0000x reference)
import jax
import jax.numpy as jnp
from jax.experimental import pallas as pl


def kernel(x_paper, x_author, edge_index_writes, edge_index_cites, params):
    raise NotImplementedError("write your pallas kernel here")


# speedup vs baseline: 1.2659x; 1.2659x over previous
import numpy as np
import jax
import jax.numpy as jnp
from jax.experimental import pallas as pl

C = 128
H = 4
D = C // H
L = 4
BLK = 512

_HM = np.zeros((C, 8), np.float32)
for _h in range(H):
    _HM[_h * D:(_h + 1) * D, _h] = 1.0
_REP = _HM.T.copy()


def _pad_rows(x, blk=BLK):
    n = x.shape[0]
    pn = ((n + blk - 1) // blk) * blk
    if pn == n:
        return x
    return jnp.pad(x, ((0, pn - n), (0, 0)))


def _full_spec(shape):
    nd = len(shape)
    return pl.BlockSpec(shape, lambda i, _nd=nd: (0,) * _nd)


def _row_spec(width):
    return pl.BlockSpec((BLK, width), lambda i: (i, 0))


def _ln(x, g, b):
    mu = x.mean(-1, keepdims=True)
    var = ((x - mu) ** 2).mean(-1, keepdims=True)
    return (x - mu) / jnp.sqrt(var + 1e-5) * g + b


def _mlp_kern(x_ref, w1, b1, g1, be1, w2, b2, g2, be2, w3, b3, o_ref):
    h = jnp.dot(x_ref[...], w1[...], preferred_element_type=jnp.float32) + b1[...]
    h = jnp.maximum(_ln(h, g1[...], be1[...]), 0.0)
    h = jnp.dot(h, w2[...], preferred_element_type=jnp.float32) + b2[...]
    h = jnp.maximum(_ln(h, g2[...], be2[...]), 0.0)
    o_ref[...] = jnp.dot(h, w3[...], preferred_element_type=jnp.float32) + b3[...]


def _mlp(x, m):
    n = x.shape[0]
    xp = _pad_rows(x)
    ops = [m["W1"], m["b1"].reshape(1, -1), m["g1"].reshape(1, -1), m["be1"].reshape(1, -1),
           m["W2"], m["b2"].reshape(1, -1), m["g2"].reshape(1, -1), m["be2"].reshape(1, -1),
           m["W3"], m["b3"].reshape(1, -1)]
    out = pl.pallas_call(
        _mlp_kern,
        grid=(xp.shape[0] // BLK,),
        in_specs=[_row_spec(xp.shape[1])] + [_full_spec(o.shape) for o in ops],
        out_specs=_row_spec(C),
        out_shape=jax.ShapeDtypeStruct((xp.shape[0], C), jnp.float32),
    )(xp, *ops)
    return out[:n]


def _mm_kern(x_ref, w_ref, b_ref, o_ref):
    o_ref[...] = jnp.dot(x_ref[...], w_ref[...], preferred_element_type=jnp.float32) + b_ref[...]


def _mm(x, w, b):
    n = x.shape[0]
    xp = _pad_rows(x)
    b2 = b.reshape(1, -1)
    out = pl.pallas_call(
        _mm_kern,
        grid=(xp.shape[0] // BLK,),
        in_specs=[_row_spec(xp.shape[1]), _full_spec(w.shape), _full_spec(b2.shape)],
        out_specs=_row_spec(w.shape[1]),
        out_shape=jax.ShapeDtypeStruct((xp.shape[0], w.shape[1]), jnp.float32),
    )(xp, w, b2)
    return out[:n]


def _alpha_kern(qd_ref, ks_ref, hm_ref, pm_ref, o_ref):
    prod = qd_ref[...] * ks_ref[...]
    o_ref[...] = jnp.dot(prod, hm_ref[...], preferred_element_type=jnp.float32) * pm_ref[...]


def _alpha(qd, ks, prel):
    pm = jnp.concatenate([prel / np.sqrt(D), jnp.zeros((8 - H,), jnp.float32)]).reshape(1, 8)
    hm = jnp.asarray(_HM)
    return pl.pallas_call(
        _alpha_kern,
        grid=(qd.shape[0] // BLK,),
        in_specs=[_row_spec(C), _row_spec(C), _full_spec(hm.shape), _full_spec(pm.shape)],
        out_specs=_row_spec(8),
        out_shape=jax.ShapeDtypeStruct((qd.shape[0], 8), jnp.float32),
    )(qd, ks, hm, pm)


def _msg_kern(vs_ref, al_ref, r_ref, o_ref):
    rep = jnp.dot(al_ref[...], r_ref[...], preferred_element_type=jnp.float32)
    o_ref[...] = vs_ref[...] * rep


def _msg(vs, alpha8):
    rep = jnp.asarray(_REP)
    return pl.pallas_call(
        _msg_kern,
        grid=(vs.shape[0] // BLK,),
        in_specs=[_row_spec(C), _row_spec(8), _full_spec(rep.shape)],
        out_specs=_row_spec(C),
        out_shape=jax.ShapeDtypeStruct((vs.shape[0], C), jnp.float32),
    )(vs, alpha8, rep)


def _post_kern(ag_ref, x_ref, w_ref, b_ref, sk_ref, o_ref):
    a = jax.nn.sigmoid(sk_ref[0, 0])
    ag = ag_ref[...]
    g = 0.5 * ag * (1.0 + jax.lax.erf(ag * np.float32(1.0 / np.sqrt(2.0))))
    o = jnp.dot(g, w_ref[...], preferred_element_type=jnp.float32) + b_ref[...]
    o_ref[...] = a * o + (1.0 - a) * x_ref[...]


def _post(aggr, x, w, b, skip):
    n = x.shape[0]
    agp = _pad_rows(aggr)
    xp = _pad_rows(x)
    b2 = b.reshape(1, -1)
    sk = skip.reshape(1, 1)
    out = pl.pallas_call(
        _post_kern,
        grid=(xp.shape[0] // BLK,),
        in_specs=[_row_spec(C), _row_spec(C), _full_spec(w.shape), _full_spec(b2.shape),
                  _full_spec(sk.shape)],
        out_specs=_row_spec(C),
        out_shape=jax.ShapeDtypeStruct((xp.shape[0], C), jnp.float32),
    )(agp, xp, w, b2, sk)
    return out[:n]


def _block_diag(rel):
    eye = jnp.eye(H, dtype=rel.dtype)
    return jnp.einsum('hde,hg->hdge', rel, eye).reshape(C, C)


def _seg_softmax(scores, seg, num):
    mx = jax.ops.segment_max(scores, seg, num_segments=num)
    mx = jnp.where(jnp.isfinite(mx), mx, 0.0)
    ex = jnp.exp(scores - mx[seg])
    den = jax.ops.segment_sum(ex, seg, num_segments=num)
    return ex / (den[seg] + 1e-16)


def _pad_idx(idx, blk=BLK):
    e = idx.shape[0]
    pe = ((e + blk - 1) // blk) * blk
    if pe == e:
        return idx
    return jnp.pad(idx, (0, pe - e))


def _edge_aggr(q_dst_tab, k_rel_tab, v_rel_tab, src, dst, prel, num_dst):
    e = src.shape[0]
    srcp = _pad_idx(src)
    dstp = _pad_idx(dst)
    qd = jnp.take(q_dst_tab, dstp, axis=0)
    ks = jnp.take(k_rel_tab, srcp, axis=0)
    vs = jnp.take(v_rel_tab, srcp, axis=0)
    logits = _alpha(qd, ks, prel)[:e, :H]
    alpha = _seg_softmax(logits, dst, num_dst)
    alpha8 = jnp.pad(alpha, ((0, srcp.shape[0] - e), (0, 8 - H)))
    msg = _msg(vs, alpha8)[:e]
    return jax.ops.segment_sum(msg, dst, num_segments=num_dst)


def _conv(xp, xa, eiw, eic, lp):
    np_, na = xp.shape[0], xa.shape[0]
    bd_a_w = _block_diag(lp["arel_writes"])
    bd_m_w = _block_diag(lp["mrel_writes"])
    bd_a_c = _block_diag(lp["arel_cites"])
    bd_m_c = _block_diag(lp["mrel_cites"])
    wa = jnp.concatenate([lp["k_author"]["W"] @ bd_a_w, lp["v_author"]["W"] @ bd_m_w], axis=1)
    ba = jnp.concatenate([lp["k_author"]["b"] @ bd_a_w, lp["v_author"]["b"] @ bd_m_w])
    proj_a = _mm(xa, wa, ba)
    k_rel_w, v_rel_w = proj_a[:, :C], proj_a[:, C:]
    wp = jnp.concatenate([lp["q_paper"]["W"], lp["k_paper"]["W"] @ bd_a_c,
                          lp["v_paper"]["W"] @ bd_m_c], axis=1)
    bp = jnp.concatenate([lp["q_paper"]["b"], lp["k_paper"]["b"] @ bd_a_c,
                          lp["v_paper"]["b"] @ bd_m_c])
    proj_p = _mm(xp, wp, bp)
    q_p, k_rel_c, v_rel_c = proj_p[:, :C], proj_p[:, C:2 * C], proj_p[:, 2 * C:]

    aggr = _edge_aggr(q_p, k_rel_w, v_rel_w, eiw[0], eiw[1], lp["prel_writes"], np_)
    aggr = aggr + _edge_aggr(q_p, k_rel_c, v_rel_c, eic[0], eic[1], lp["prel_cites"], np_)

    new_p = _post(aggr, xp, lp["a_paper"]["W"], lp["a_paper"]["b"], lp["skip_paper"])
    zeros_a = jnp.zeros((na, C), jnp.float32)
    new_a = _post(zeros_a, xa, lp["a_author"]["W"], lp["a_author"]["b"], lp["skip_author"])
    return new_p, new_a


def kernel(x_paper, x_author, edge_index_writes, edge_index_cites, params):
    xp = _mlp(x_paper, params["in_paper"])
    xa = _mlp(x_author, params["in_author"])
    for l in range(L):
        xp, xa = _conv(xp, xa, edge_index_writes, edge_index_cites, params["conv%d" % l])
    out_p = _mm(xp, params["out_paper"]["W"], params["out_paper"]["b"])
    out_a = _mm(xa, params["out_author"]["W"], params["out_author"]["b"])
    return (out_p, out_a)
